# baseline (speedup 1.0000x reference)
import functools

import jax
import jax.numpy as jnp
from jax import lax
from jax.experimental import pallas as pl
from jax.experimental.pallas import tpu as pltpu
from jax.experimental.pallas import tpu_sc as plsc

B, T, D, E, C = 4096, 16, 1024, 8, 256
BLK = 256
S_SC = 1024
H_TC = B - S_SC
NW = 32
CG = 16


def _pool_mm_body(sid_ref, x_ref, w_ref, b_ref, out_ref):
    xp = jnp.sum(x_ref[...], axis=1) * (1.0 / T)
    acc = lax.dot_general(
        xp, w_ref[...],
        dimension_numbers=(((1,), (1,)), ((), ())),
        preferred_element_type=jnp.float32,
    )
    sid = sid_ref[0, 0, :]
    out = jnp.zeros((BLK, C), dtype=jnp.float32)
    for e in range(E):
        mask = (sid == e).astype(jnp.float32)[:, None]
        out = out + mask * (acc[:, e * C:(e + 1) * C] + b_ref[e, :][None, :])
    out_ref[...] = out


def _mm_body(sid_ref, xp_ref, w_ref, b_ref, out_ref):
    acc = lax.dot_general(
        xp_ref[...], w_ref[...],
        dimension_numbers=(((1,), (1,)), ((), ())),
        preferred_element_type=jnp.float32,
    )
    sid = sid_ref[0, 0, :]
    out = jnp.zeros((BLK, C), dtype=jnp.float32)
    for e in range(E):
        mask = (sid == e).astype(jnp.float32)[:, None]
        out = out + mask * (acc[:, e * C:(e + 1) * C] + b_ref[e, :][None, :])
    out_ref[...] = out


def _sc_pool_body(x_hbm, out_hbm, buf, obuf):
    wid = lax.axis_index("s") * 2 + lax.axis_index("c")
    n = S_SC // NW
    base = H_TC + wid * n

    def sample_body(j, carry):
        i = base + j
        pltpu.sync_copy(x_hbm.at[i], buf)

        def cg_body(cg, c2):
            off = cg * CG
            s = buf[0, pl.ds(off, CG)]
            for r in range(1, T):
                s = s + buf[r, pl.ds(off, CG)]
            obuf[pl.ds(off, CG)] = s * (1.0 / T)
            return c2

        lax.fori_loop(0, D // CG, cg_body, 0)
        pltpu.sync_copy(obuf, out_hbm.at[i - H_TC])
        return carry

    lax.fori_loop(0, n, sample_body, 0)


_sc_pool = functools.partial(
    pl.kernel,
    mesh=plsc.VectorSubcoreMesh(core_axis_name="c", subcore_axis_name="s"),
    out_type=jax.ShapeDtypeStruct((S_SC, D), jnp.float32),
    scratch_types=[
        pltpu.VMEM((T, D), jnp.float32),
        pltpu.VMEM((D,), jnp.float32),
    ],
)(_sc_pool_body)


def kernel(x, system_id, W, b):
    nblk_h = H_TC // BLK
    nblk_t = S_SC // BLK
    sid3 = system_id.astype(jnp.int32).reshape(B // BLK, 1, BLK)
    wcat = W.reshape(E * C, D)

    xp_tail = _sc_pool(x)

    out_head = pl.pallas_call(
        _pool_mm_body,
        grid=(nblk_h,),
        in_specs=[
            pl.BlockSpec((1, 1, BLK), lambda g: (g, 0, 0)),
            pl.BlockSpec((BLK, T, D), lambda g: (g, 0, 0)),
            pl.BlockSpec((E * C, D), lambda g: (0, 0)),
            pl.BlockSpec((E, C), lambda g: (0, 0)),
        ],
        out_specs=pl.BlockSpec((BLK, C), lambda g: (g, 0)),
        out_shape=jax.ShapeDtypeStruct((H_TC, C), jnp.float32),
        compiler_params=pltpu.CompilerParams(
            dimension_semantics=("arbitrary",),
        ),
    )(sid3, x, wcat, b)

    out_tail = pl.pallas_call(
        _mm_body,
        grid=(nblk_t,),
        in_specs=[
            pl.BlockSpec((1, 1, BLK), lambda g: (g + H_TC // BLK, 0, 0)),
            pl.BlockSpec((BLK, D), lambda g: (g, 0)),
            pl.BlockSpec((E * C, D), lambda g: (0, 0)),
            pl.BlockSpec((E, C), lambda g: (0, 0)),
        ],
        out_specs=pl.BlockSpec((BLK, C), lambda g: (g, 0)),
        out_shape=jax.ShapeDtypeStruct((S_SC, C), jnp.float32),
        compiler_params=pltpu.CompilerParams(
            dimension_semantics=("arbitrary",),
        ),
    )(sid3, xp_tail, wcat, b)

    return jnp.concatenate([out_head, out_tail], axis=0)


# speedup vs baseline: 1.2023x; 1.2023x over previous
import functools

import jax
import jax.numpy as jnp
from jax import lax
from jax.experimental import pallas as pl
from jax.experimental.pallas import tpu as pltpu
from jax.experimental.pallas import tpu_sc as plsc

B, T, D, E, C = 4096, 16, 1024, 8, 256
BLK = 256
S_SC = 1024
H_TC = B - S_SC
NW = 32
CG = 16
NB = 4


def _pool_mm_body(sid_ref, x_ref, w_ref, b_ref, out_ref):
    xp = jnp.sum(x_ref[...], axis=1) * (1.0 / T)
    acc = lax.dot_general(
        xp, w_ref[...],
        dimension_numbers=(((1,), (1,)), ((), ())),
        preferred_element_type=jnp.float32,
    )
    sid = sid_ref[0, 0, :]
    out = jnp.zeros((BLK, C), dtype=jnp.float32)
    for e in range(E):
        mask = (sid == e).astype(jnp.float32)[:, None]
        out = out + mask * (acc[:, e * C:(e + 1) * C] + b_ref[e, :][None, :])
    out_ref[...] = out


def _mm_body(sid_ref, xp_ref, w_ref, b_ref, _o_ref, out_ref):
    acc = lax.dot_general(
        xp_ref[...], w_ref[...],
        dimension_numbers=(((1,), (1,)), ((), ())),
        preferred_element_type=jnp.float32,
    )
    sid = sid_ref[0, 0, :]
    out = jnp.zeros((BLK, C), dtype=jnp.float32)
    for e in range(E):
        mask = (sid == e).astype(jnp.float32)[:, None]
        out = out + mask * (acc[:, e * C:(e + 1) * C] + b_ref[e, :][None, :])
    out_ref[...] = out


def _sc_pool_body(x_hbm, out_hbm, buf, obuf, *sems):
    in_sems = sems[:NB]
    out_sems = sems[NB:]
    wid = lax.axis_index("s") * 2 + lax.axis_index("c")
    n = S_SC // NW
    base = H_TC + wid * n

    for k in range(NB):
        pltpu.make_async_copy(x_hbm.at[base + k], buf.at[k], in_sems[k]).start()

    def outer(it, carry):
        j0 = it * NB
        for k in range(NB):
            j = j0 + k
            i = base + j
            pltpu.make_async_copy(x_hbm.at[i], buf.at[k], in_sems[k]).wait()

            def cg_body(cg, c2):
                off = cg * CG
                s = buf[k, 0, pl.ds(off, CG)]
                for r in range(1, T):
                    s = s + buf[k, r, pl.ds(off, CG)]
                obuf[k, pl.ds(off, CG)] = s * (1.0 / T)
                return c2

            @pl.when(it > 0)
            def _wait_out():
                pltpu.make_async_copy(
                    obuf.at[k], out_hbm.at[i - H_TC], out_sems[k]).wait()

            lax.fori_loop(0, D // CG, cg_body, 0)
            pltpu.make_async_copy(
                obuf.at[k], out_hbm.at[i - H_TC], out_sems[k]).start()

            @pl.when(j + NB < n)
            def _refill():
                pltpu.make_async_copy(
                    x_hbm.at[i + NB], buf.at[k], in_sems[k]).start()

        return carry

    lax.fori_loop(0, n // NB, outer, 0)

    for k in range(NB):
        pltpu.make_async_copy(
            obuf.at[k], out_hbm.at[base - H_TC + n - NB + k], out_sems[k]).wait()


_sc_pool = functools.partial(
    pl.kernel,
    mesh=plsc.VectorSubcoreMesh(core_axis_name="c", subcore_axis_name="s"),
    out_type=jax.ShapeDtypeStruct((S_SC, D), jnp.float32),
    scratch_types=(
        [pltpu.VMEM((NB, T, D), jnp.float32), pltpu.VMEM((NB, D), jnp.float32)]
        + [pltpu.SemaphoreType.DMA] * (2 * NB)
    ),
)(_sc_pool_body)


def kernel(x, system_id, W, b):
    nblk_h = H_TC // BLK
    nblk_t = S_SC // BLK
    sid3 = system_id.astype(jnp.int32).reshape(B // BLK, 1, BLK)
    wcat = W.reshape(E * C, D)

    xp_tail = _sc_pool(x)

    out_head = pl.pallas_call(
        _pool_mm_body,
        grid=(nblk_h,),
        in_specs=[
            pl.BlockSpec((1, 1, BLK), lambda g: (g, 0, 0)),
            pl.BlockSpec((BLK, T, D), lambda g: (g, 0, 0)),
            pl.BlockSpec((E * C, D), lambda g: (0, 0)),
            pl.BlockSpec((E, C), lambda g: (0, 0)),
        ],
        out_specs=pl.BlockSpec((BLK, C), lambda g: (g, 0)),
        out_shape=jax.ShapeDtypeStruct((B, C), jnp.float32),
        compiler_params=pltpu.CompilerParams(
            dimension_semantics=("arbitrary",),
        ),
    )(sid3, x, wcat, b)

    out = pl.pallas_call(
        _mm_body,
        grid=(nblk_t,),
        in_specs=[
            pl.BlockSpec((1, 1, BLK), lambda g: (g + H_TC // BLK, 0, 0)),
            pl.BlockSpec((BLK, D), lambda g: (g, 0)),
            pl.BlockSpec((E * C, D), lambda g: (0, 0)),
            pl.BlockSpec((E, C), lambda g: (0, 0)),
            pl.BlockSpec((BLK, C), lambda g: (g + H_TC // BLK, 0)),
        ],
        out_specs=pl.BlockSpec((BLK, C), lambda g: (g + H_TC // BLK, 0)),
        out_shape=jax.ShapeDtypeStruct((B, C), jnp.float32),
        input_output_aliases={4: 0},
        compiler_params=pltpu.CompilerParams(
            dimension_semantics=("arbitrary",),
        ),
    )(sid3, xp_tail, wcat, b, out_head)

    return out


# speedup vs baseline: 1.5958x; 1.3273x over previous
import jax
import jax.numpy as jnp
from jax import lax
from jax.experimental import pallas as pl
from jax.experimental.pallas import tpu as pltpu

B, T, D, E, C = 4096, 16, 1024, 8, 256
BLK = 256


def _fused_body(sid_ref, x_ref, w_ref, b_ref, out_ref):
    xp = jnp.sum(x_ref[...], axis=1) * (1.0 / T)
    acc = lax.dot_general(
        xp, w_ref[...],
        dimension_numbers=(((1,), (1,)), ((), ())),
        preferred_element_type=jnp.float32,
    )
    sid = sid_ref[0, 0, :]
    out = jnp.zeros((BLK, C), dtype=jnp.float32)
    for e in range(E):
        mask = (sid == e).astype(jnp.float32)[:, None]
        out = out + mask * (acc[:, e * C:(e + 1) * C] + b_ref[e, :][None, :])
    out_ref[...] = out


def kernel(x, system_id, W, b):
    nblk = B // BLK
    sid3 = system_id.astype(jnp.int32).reshape(nblk, 1, BLK)
    wcat = W.reshape(E * C, D)
    return pl.pallas_call(
        _fused_body,
        grid=(nblk,),
        in_specs=[
            pl.BlockSpec((1, 1, BLK), lambda g: (g, 0, 0)),
            pl.BlockSpec((BLK, T, D), lambda g: (g, 0, 0)),
            pl.BlockSpec((E * C, D), lambda g: (0, 0)),
            pl.BlockSpec((E, C), lambda g: (0, 0)),
        ],
        out_specs=pl.BlockSpec((BLK, C), lambda g: (g, 0)),
        out_shape=jax.ShapeDtypeStruct((B, C), jnp.float32),
        compiler_params=pltpu.CompilerParams(
            dimension_semantics=("arbitrary",),
        ),
    )(sid3, x, wcat, b)
